# baseline (speedup 1.0000x reference)
import jax
import jax.numpy as jnp
from jax import lax
from jax.experimental import pallas as pl
from jax.experimental.pallas import tpu as pltpu
from jax.experimental.pallas import tpu_sc as plsc

NC = 2
NS = 16
L = 16
NW = NC * NS

B = 16384
D = 32
BPW = B // NW
CHUNK = 128
NCHUNK = BPW // CHUNK


def _fire(tables, idx_v, bufs, sem, d):
    for j in range(6):
        strip = tables[j % 3].at[d]
        for c in range(NCHUNK):
            pltpu.make_async_copy(
                strip.at[idx_v.at[j, pl.ds(c * CHUNK, CHUNK)]],
                bufs[j].at[pl.ds(c * CHUNK, CHUNK)],
                sem,
            ).start()


def _drain(tables, idx_v, bufs, sem, d):
    for j in range(6):
        strip = tables[j % 3].at[d]
        for c in range(NCHUNK):
            pltpu.make_async_copy(
                strip.at[idx_v.at[j, pl.ds(c * CHUNK, CHUNK)]],
                bufs[j].at[pl.ds(c * CHUNK, CHUNK)],
                sem,
            ).wait()


def _accumulate(bufs, acc_v, first):
    hp, rp, tp, hn, rn, tn = bufs
    for k in range(BPW // L):
        s = pl.ds(k * L, L)
        dp = jnp.abs(hp[s] + rp[s] - tp[s])
        dn = jnp.abs(hn[s] + rn[s] - tn[s])
        contrib = dp - dn
        if first:
            acc_v[s] = contrib
        else:
            acc_v[s] = acc_v[s] + contrib


def _sc_kernel(idx6, event_t, edgetype_t, attrib_t, out_hbm,
               idx_v,
               a0, a1, a2, a3, a4, a5,
               b0, b1, b2, b3, b4, b5,
               acc_v, sema, semb):
    wid = lax.axis_index("s") * NC + lax.axis_index("c")
    base = wid * BPW

    pltpu.sync_copy(idx6.at[:, wid], idx_v)

    tables = (event_t, edgetype_t, attrib_t)
    bufsa = (a0, a1, a2, a3, a4, a5)
    bufsb = (b0, b1, b2, b3, b4, b5)

    _fire(tables, idx_v, bufsa, sema, 0)
    _fire(tables, idx_v, bufsb, semb, 1)
    _drain(tables, idx_v, bufsa, sema, 0)
    _accumulate(bufsa, acc_v, first=True)

    def pair_body(k, _):
        da = 2 * k + 2
        db = 2 * k + 1

        @pl.when(k < (D // 2 - 1))
        def _():
            _fire(tables, idx_v, bufsa, sema, da)

        _drain(tables, idx_v, bufsb, semb, db)

        @pl.when(k < (D // 2 - 1))
        def _():
            _fire(tables, idx_v, bufsb, semb, db + 2)

        _accumulate(bufsb, acc_v, first=False)

        @pl.when(k < (D // 2 - 1))
        def _():
            _drain(tables, idx_v, bufsa, sema, da)
            _accumulate(bufsa, acc_v, first=False)

        return 0

    lax.fori_loop(0, D // 2, pair_body, 0)

    zeros = jnp.zeros((L,), jnp.float32)
    ones = jnp.full((L,), 1.0, jnp.float32)
    for k in range(BPW // L):
        s = pl.ds(k * L, L)
        acc_v[s] = jnp.maximum(zeros, acc_v[s] + ones)

    pltpu.sync_copy(acc_v, out_hbm.at[pl.ds(base, BPW)])


@jax.jit
def _run(idx6, event_t, edgetype_t, attrib_t):
    mesh = plsc.VectorSubcoreMesh(core_axis_name="c", subcore_axis_name="s")
    fbuf = pltpu.VMEM((BPW,), jnp.float32)
    return pl.kernel(
        _sc_kernel,
        out_type=jax.ShapeDtypeStruct((B,), jnp.float32),
        mesh=mesh,
        compiler_params=pltpu.CompilerParams(
            needs_layout_passes=False, use_tc_tiling_on_sc=False
        ),
        scratch_types=[
            pltpu.VMEM((6, BPW), jnp.int32),
            fbuf, fbuf, fbuf, fbuf, fbuf, fbuf,
            fbuf, fbuf, fbuf, fbuf, fbuf, fbuf,
            fbuf,
            pltpu.SemaphoreType.DMA,
            pltpu.SemaphoreType.DMA,
        ],
    )(idx6, event_t, edgetype_t, attrib_t)


def kernel(pos_triplets, neg_triplets, event_em, edgetype_em, attrib_em):
    idx6 = jnp.concatenate(
        [pos_triplets.T, neg_triplets.T], axis=0
    ).reshape(6, NW, BPW)
    return _run(idx6, event_em.T, edgetype_em.T, attrib_em.T)


# speedup vs baseline: 6.0259x; 6.0259x over previous
import jax
import jax.numpy as jnp
from jax import lax
from jax.experimental import pallas as pl
from jax.experimental.pallas import tpu as pltpu
from jax.experimental.pallas import tpu_sc as plsc

NC = 2
NS = 16
L = 16
NW = NC * NS

B = 16384
D = 32
BPW = B // NW
CHUNK = 128
NCHUNK = BPW // CHUNK
NGROUP = BPW // L


def _sc_kernel(idx6, event_em, edgetype_em, attrib_em, out_hbm,
               idx_v, ph, pr, pt, nh, nr, nt, out_v, sem):
    wid = lax.axis_index("s") * NC + lax.axis_index("c")
    base = wid * BPW

    pltpu.sync_copy(idx6.at[:, wid], idx_v)

    tables = (event_em, edgetype_em, attrib_em,
              event_em, edgetype_em, attrib_em)
    bufs = (ph, pr, pt, nh, nr, nt)

    copies = []
    for j in range(6):
        for c in range(NCHUNK):
            cp = pltpu.make_async_copy(
                tables[j].at[idx_v.at[j, pl.ds(c * CHUNK, CHUNK)]],
                bufs[j].at[pl.ds(c * CHUNK, CHUNK), :],
                sem,
            )
            cp.start()
            copies.append(cp)
    for cp in copies:
        cp.wait()

    def margin_diff(b):
        s0 = pl.ds(0, L)
        s1 = pl.ds(L, L)
        dp = jnp.abs(ph[b, s0] + pr[b, s0] - pt[b, s0]) + jnp.abs(
            ph[b, s1] + pr[b, s1] - pt[b, s1]
        )
        dn = jnp.abs(nh[b, s0] + nr[b, s0] - nt[b, s0]) + jnp.abs(
            nh[b, s1] + nr[b, s1] - nt[b, s1]
        )
        return jnp.sum(dp - dn)

    lane = lax.iota(jnp.int32, L)
    zeros = jnp.zeros((L,), jnp.float32)
    ones = jnp.full((L,), 1.0, jnp.float32)

    def group_body(g, _):
        vloss = zeros
        for u in range(L):
            sc = margin_diff(g * L + u)
            vloss = jnp.where(lane == u, lax.broadcast(sc, (L,)), vloss)
        out_v[pl.ds(g * L, L)] = jnp.maximum(zeros, vloss + ones)
        return 0

    lax.fori_loop(0, NGROUP, group_body, 0)

    pltpu.sync_copy(out_v, out_hbm.at[pl.ds(base, BPW)])


@jax.jit
def _run(idx6, event_em, edgetype_em, attrib_em):
    mesh = plsc.VectorSubcoreMesh(core_axis_name="c", subcore_axis_name="s")
    return pl.kernel(
        _sc_kernel,
        out_type=jax.ShapeDtypeStruct((B,), jnp.float32),
        mesh=mesh,
        compiler_params=pltpu.CompilerParams(
            needs_layout_passes=False, use_tc_tiling_on_sc=False
        ),
        scratch_types=[
            pltpu.VMEM((6, BPW), jnp.int32),
            pltpu.VMEM((BPW, D), jnp.float32),
            pltpu.VMEM((BPW, D), jnp.float32),
            pltpu.VMEM((BPW, D), jnp.float32),
            pltpu.VMEM((BPW, D), jnp.float32),
            pltpu.VMEM((BPW, D), jnp.float32),
            pltpu.VMEM((BPW, D), jnp.float32),
            pltpu.VMEM((BPW,), jnp.float32),
            pltpu.SemaphoreType.DMA,
        ],
    )(idx6, event_em, edgetype_em, attrib_em)


def kernel(pos_triplets, neg_triplets, event_em, edgetype_em, attrib_em):
    idx6 = jnp.concatenate(
        [pos_triplets.T, neg_triplets.T], axis=0
    ).reshape(6, NW, BPW)
    return _run(idx6, event_em, edgetype_em, attrib_em)
